# Initial kernel scaffold; baseline (speedup 1.0000x reference)
import jax
import jax.numpy as jnp
from jax.experimental import pallas as pl


def kernel(x, batch, W_aggr, W_out, b_out):
    raise NotImplementedError("write your pallas kernel here")


# speedup vs baseline: 4.3743x; 4.3743x over previous
import functools

import jax
import jax.numpy as jnp
from jax.experimental import pallas as pl
from jax.experimental.pallas import tpu as pltpu

NUM_SEG = 512
R = 512
NEG_INF = float("-inf")


def _fused_body(batch_sm, x_ref, brow_ref, bcol_ref, w34_ref, wsum_ref,
                wout_ref, bout_ref, out_ref, sacc, macc, cacc, *, nblocks):
    step = pl.program_id(0)

    @pl.when(step == 0)
    def _init():
        sacc[...] = jnp.zeros_like(sacc)
        macc[...] = jnp.full_like(macc, NEG_INF)
        cacc[...] = jnp.zeros_like(cacc)

    x = x_ref[...]
    h34 = jnp.dot(x, w34_ref[...],
                  preferred_element_type=jnp.float32)
    a = h34.shape[1] // 2
    h3 = h34[:, :a]
    q = h34[:, a:] * h34[:, a:]

    b_row = brow_ref[0]
    b_col = bcol_ref[0]

    ot = (jax.lax.broadcasted_iota(jnp.int32, (NUM_SEG, R), 0)
          == b_row).astype(jnp.float32)
    d = jnp.concatenate([x, q], axis=1)
    sacc[...] += jnp.dot(ot, d, preferred_element_type=jnp.float32)
    cacc[...] += jnp.dot(ot, jnp.ones((R, 1), jnp.float32),
                         preferred_element_type=jnp.float32)

    lo = batch_sm[0, 0, 0]
    hi = jnp.minimum(batch_sm[0, 0, R - 1], NUM_SEG - 1)

    def _visit(j, carry):
        s = lo + j
        mask = b_col == s
        m = jnp.max(jnp.where(mask, h3, NEG_INF), axis=0, keepdims=True)
        macc[pl.ds(s, 1), :] = jnp.maximum(macc[pl.ds(s, 1), :], m)
        return carry

    jax.lax.fori_loop(0, hi - lo + 1, _visit, 0)

    @pl.when(step == nblocks - 1)
    def _finish():
        h = x_ref.shape[1]
        s_all = sacc[...]
        sx = s_all[:, :h]
        sq = s_all[:, h:]
        invc = 1.0 / jnp.maximum(cacc[...], 1.0)
        t = jnp.dot(sx, wsum_ref[...],
                    preferred_element_type=jnp.float32)
        z1 = t[:, :a]
        z2 = t[:, a:2 * a] * invc
        mean4 = t[:, 2 * a:] * invc
        mean2 = sq * invc
        var = mean2 - mean4 * mean4
        z4 = jnp.sqrt(jnp.maximum(var, 0.0) + 1e-5)
        m = macc[...]
        z3 = jnp.where(jnp.isfinite(m), m, 0.0)
        z = jnp.concatenate([z1, z2, z3, z4], axis=1)
        out_ref[...] = (jnp.dot(z, wout_ref[...],
                                preferred_element_type=jnp.float32)
                        + bout_ref[...])


def kernel(x, batch, W_aggr, W_out, b_out):
    n, h = x.shape
    a4 = W_aggr.shape[1]
    a = a4 // 4

    nblocks = (n + R - 1) // R
    n_pad = nblocks * R
    batch = batch.astype(jnp.int32)
    if n_pad != n:
        x = jnp.pad(x, ((0, n_pad - n), (0, 0)))
        batch = jnp.pad(batch, (0, n_pad - n), constant_values=NUM_SEG)

    b_row = batch.reshape(nblocks, 1, R)
    b_col = batch.reshape(nblocks, R, 1)
    b_sm = batch.reshape(nblocks, 1, R)

    w34 = W_aggr[:, 2 * a:]
    wsum = jnp.concatenate([W_aggr[:, :2 * a], W_aggr[:, 3 * a:]], axis=1)
    bout2 = b_out.reshape(1, h)

    grid = (nblocks,)
    body = functools.partial(_fused_body, nblocks=nblocks)
    out = pl.pallas_call(
        body,
        grid=grid,
        in_specs=[
            pl.BlockSpec((1, 1, R), lambda i: (i, 0, 0),
                         memory_space=pltpu.SMEM),
            pl.BlockSpec((R, h), lambda i: (i, 0)),
            pl.BlockSpec((1, 1, R), lambda i: (i, 0, 0)),
            pl.BlockSpec((1, R, 1), lambda i: (i, 0, 0)),
            pl.BlockSpec((h, 2 * a), lambda i: (0, 0)),
            pl.BlockSpec((h, 3 * a), lambda i: (0, 0)),
            pl.BlockSpec((a4, h), lambda i: (0, 0)),
            pl.BlockSpec((1, h), lambda i: (0, 0)),
        ],
        out_specs=pl.BlockSpec((NUM_SEG, h), lambda i: (0, 0)),
        out_shape=jax.ShapeDtypeStruct((NUM_SEG, h), jnp.float32),
        scratch_shapes=[
            pltpu.VMEM((NUM_SEG, h + a), jnp.float32),
            pltpu.VMEM((NUM_SEG, a), jnp.float32),
            pltpu.VMEM((NUM_SEG, 1), jnp.float32),
        ],
        compiler_params=pltpu.CompilerParams(
            dimension_semantics=("arbitrary",),
        ),
    )(b_sm, x, b_row, b_col, w34, wsum, W_out, bout2)
    return out


# speedup vs baseline: 4.9118x; 1.1229x over previous
import functools

import jax
import jax.numpy as jnp
from jax.experimental import pallas as pl
from jax.experimental.pallas import tpu as pltpu

NUM_SEG = 512
R = 512
SW = 64
APAD = NUM_SEG + 2 * SW
NEG_INF = float("-inf")


def _fused_body(batch_sm, x_ref, brow_ref, bcol_ref, w34_ref, wsum_ref,
                wout_ref, bout_ref, out_ref, sacc, macc, cacc, *, nblocks):
    step = pl.program_id(0)

    @pl.when(step == 0)
    def _init():
        sacc[...] = jnp.zeros_like(sacc)
        macc[...] = jnp.full_like(macc, NEG_INF)
        cacc[...] = jnp.zeros_like(cacc)

    x = x_ref[...]
    h34 = jnp.dot(x, w34_ref[...],
                  preferred_element_type=jnp.float32)
    a = h34.shape[1] // 2
    h3 = h34[:, :a]
    q = h34[:, a:] * h34[:, a:]

    b_row = brow_ref[0]
    b_col = bcol_ref[0]
    d = jnp.concatenate([x, q], axis=1)

    lo = batch_sm[0, 0, 0]
    hi = jnp.minimum(batch_sm[0, 0, R - 1], NUM_SEG - 1)

    base0 = (lo // 8) * 8
    nwin = (hi - base0) // SW + 1

    def _win(w, carry):
        base = base0 + w * SW
        ots = (jax.lax.broadcasted_iota(jnp.int32, (SW, R), 0) + base
               == b_row).astype(jnp.float32)
        sacc[pl.ds(base, SW), :] += jnp.dot(
            ots, d, preferred_element_type=jnp.float32)
        cacc[pl.ds(base, SW), :] += jnp.dot(
            ots, jnp.ones((R, 1), jnp.float32),
            preferred_element_type=jnp.float32)
        return carry

    jax.lax.fori_loop(0, nwin, _win, 0)

    def _visit(j, carry):
        s = lo + j
        mask = b_col == s
        m = jnp.max(jnp.where(mask, h3, NEG_INF), axis=0, keepdims=True)
        macc[pl.ds(s, 1), :] = jnp.maximum(macc[pl.ds(s, 1), :], m)
        return carry

    jax.lax.fori_loop(0, hi - lo + 1, _visit, 0)

    @pl.when(step == nblocks - 1)
    def _finish():
        h = x_ref.shape[1]
        s_all = sacc[:NUM_SEG, :]
        sx = s_all[:, :h]
        sq = s_all[:, h:]
        invc = 1.0 / jnp.maximum(cacc[:NUM_SEG, :], 1.0)
        t = jnp.dot(sx, wsum_ref[...],
                    preferred_element_type=jnp.float32)
        z1 = t[:, :a]
        z2 = t[:, a:2 * a] * invc
        mean4 = t[:, 2 * a:] * invc
        mean2 = sq * invc
        var = mean2 - mean4 * mean4
        z4 = jnp.sqrt(jnp.maximum(var, 0.0) + 1e-5)
        m = macc[...]
        z3 = jnp.where(jnp.isfinite(m), m, 0.0)
        z = jnp.concatenate([z1, z2, z3, z4], axis=1)
        out_ref[...] = (jnp.dot(z, wout_ref[...],
                                preferred_element_type=jnp.float32)
                        + bout_ref[...])


def kernel(x, batch, W_aggr, W_out, b_out):
    n, h = x.shape
    a4 = W_aggr.shape[1]
    a = a4 // 4

    nblocks = (n + R - 1) // R
    n_pad = nblocks * R
    batch = batch.astype(jnp.int32)
    if n_pad != n:
        x = jnp.pad(x, ((0, n_pad - n), (0, 0)))
        batch = jnp.pad(batch, (0, n_pad - n), constant_values=NUM_SEG)

    b_row = batch.reshape(nblocks, 1, R)
    b_col = batch.reshape(nblocks, R, 1)
    b_sm = batch.reshape(nblocks, 1, R)

    w34 = W_aggr[:, 2 * a:]
    wsum = jnp.concatenate([W_aggr[:, :2 * a], W_aggr[:, 3 * a:]], axis=1)
    bout2 = b_out.reshape(1, h)

    grid = (nblocks,)
    body = functools.partial(_fused_body, nblocks=nblocks)
    out = pl.pallas_call(
        body,
        grid=grid,
        in_specs=[
            pl.BlockSpec((1, 1, R), lambda i: (i, 0, 0),
                         memory_space=pltpu.SMEM),
            pl.BlockSpec((R, h), lambda i: (i, 0)),
            pl.BlockSpec((1, 1, R), lambda i: (i, 0, 0)),
            pl.BlockSpec((1, R, 1), lambda i: (i, 0, 0)),
            pl.BlockSpec((h, 2 * a), lambda i: (0, 0)),
            pl.BlockSpec((h, 3 * a), lambda i: (0, 0)),
            pl.BlockSpec((a4, h), lambda i: (0, 0)),
            pl.BlockSpec((1, h), lambda i: (0, 0)),
        ],
        out_specs=pl.BlockSpec((NUM_SEG, h), lambda i: (0, 0)),
        out_shape=jax.ShapeDtypeStruct((NUM_SEG, h), jnp.float32),
        scratch_shapes=[
            pltpu.VMEM((APAD, h + a), jnp.float32),
            pltpu.VMEM((NUM_SEG, a), jnp.float32),
            pltpu.VMEM((APAD, 1), jnp.float32),
        ],
        compiler_params=pltpu.CompilerParams(
            dimension_semantics=("arbitrary",),
        ),
    )(b_sm, x, b_row, b_col, w34, wsum, W_out, bout2)
    return out
